# baseline (speedup 1.0000x reference)
import functools

import jax
import jax.numpy as jnp
from jax import lax
from jax.experimental import pallas as pl
from jax.experimental.pallas import tpu as pltpu
from jax.experimental.pallas import tpu_sc as plsc

RADIUS = 0.05
NSC = 16384


def _pack_kernel(p_ref, osc_ref, otc_ref, *, nsc):
    pt = p_ref[...]
    pnn = -0.5 * (pt[0:1, :] * pt[0:1, :] + pt[1:2, :] * pt[1:2, :]
                  + pt[2:3, :] * pt[2:3, :])
    pk = jnp.concatenate([pt, pnn], axis=0)
    osc_ref[...] = pk[:, :nsc]
    otc_ref[...] = jnp.broadcast_to(pk[:, None, nsc:], otc_ref.shape)


def _nn_tc_kernel(x_ref, pall_ref, o_ref, xb0_ref, xb1_ref, xb2_ref,
                  acc_ref, *, pc, n, tm):
    xt = x_ref[...]
    xb0_ref[...] = jnp.broadcast_to(xt[:, 0:1], (tm, 128))
    xb1_ref[...] = jnp.broadcast_to(xt[:, 1:2], (tm, 128))
    xb2_ref[...] = jnp.broadcast_to(xt[:, 2:3], (tm, 128))
    acc_ref[...] = jnp.full((tm, 128), -jnp.inf, jnp.float32)

    nq = 2

    def qsloop(qs, _):
        rq = qs * (8 * nq)
        xs = []
        for b in range(nq):
            r = rq + 8 * b
            xs.append((xb0_ref[pl.ds(r, 8), :],
                       xb1_ref[pl.ds(r, 8), :],
                       xb2_ref[pl.ds(r, 8), :]))

        def ploop(k, accs):
            base = k * pc
            p0 = pall_ref[0, :, pl.ds(base, pc)]
            p1 = pall_ref[1, :, pl.ds(base, pc)]
            p2 = pall_ref[2, :, pl.ds(base, pc)]
            pnn = pall_ref[3, :, pl.ds(base, pc)]
            out = []
            for b in range(nq):
                x0, x1, x2 = xs[b]
                a = accs[b]
                for j in range(pc // 128):
                    s = slice(j * 128, (j + 1) * 128)
                    u = x0 * p0[:, s] + (x1 * p1[:, s]
                                         + (x2 * p2[:, s] + pnn[:, s]))
                    a = jnp.maximum(a, u)
                out.append(a)
            return tuple(out)

        neg = jnp.full((8, 128), -jnp.inf, jnp.float32)
        accs = lax.fori_loop(0, n // pc, ploop, (neg,) * nq, unroll=2)
        for b in range(nq):
            acc_ref[pl.ds(rq + 8 * b, 8), :] = accs[b]
        return 0

    lax.fori_loop(0, tm // (8 * nq), qsloop, 0)
    o_ref[:, 0] = jnp.max(acc_ref[...], axis=1)


def _combine_kernel(x_ref, utc_ref, usc_ref, o_ref):
    xt = x_ref[...]
    xn = (xt[:, 0] * xt[:, 0] + xt[:, 1] * xt[:, 1] + xt[:, 2] * xt[:, 2])
    u = jnp.maximum(utc_ref[:, 0], jnp.max(usc_ref[...], axis=1))
    d2 = jnp.maximum(xn - 2.0 * u, 0.0)
    o_ref[:, 0] = jnp.sqrt(d2) - RADIUS


def _make_sc_kernel(m, nsc):
    mq = m // 32
    mesh = plsc.VectorSubcoreMesh(core_axis_name="c", subcore_axis_name="s")

    @functools.partial(
        pl.kernel, mesh=mesh,
        out_type=jax.ShapeDtypeStruct((m, 16), jnp.float32),
        scratch_types=[
            pltpu.VMEM((4, mq), jnp.float32),
            pltpu.VMEM((4, nsc), jnp.float32),
            pltpu.VMEM((mq, 16), jnp.float32),
        ],
    )
    def sck(x_hbm, p_hbm, out_hbm, x_v, p_v, out_v):
        wid = lax.axis_index("s") * 2 + lax.axis_index("c")
        base = wid * mq
        pltpu.sync_copy(x_hbm.at[:, pl.ds(base, mq)], x_v)
        pltpu.sync_copy(p_hbm, p_v)

        def qloop(qs, _):
            q = qs * 16
            x0v = x_v[0, pl.ds(q, 16)]
            x1v = x_v[1, pl.ds(q, 16)]
            x2v = x_v[2, pl.ds(q, 16)]
            for sub in range(4):
                xs = []
                for i in range(4):
                    e = sub * 4 + i
                    xs.append((jnp.broadcast_to(x0v[e], (16,)),
                               jnp.broadcast_to(x1v[e], (16,)),
                               jnp.broadcast_to(x2v[e], (16,))))

                def jloop(j, accs):
                    c = j * 16
                    p0 = p_v[0, pl.ds(c, 16)]
                    p1 = p_v[1, pl.ds(c, 16)]
                    p2 = p_v[2, pl.ds(c, 16)]
                    pnn = p_v[3, pl.ds(c, 16)]
                    return tuple(
                        jnp.maximum(a, b0 * p0 + (b1 * p1 + (b2 * p2 + pnn)))
                        for a, (b0, b1, b2) in zip(accs, xs))

                neg = jnp.full((16,), -jnp.inf, jnp.float32)
                accs = lax.fori_loop(0, nsc // 16, jloop,
                                     (neg, neg, neg, neg))
                for i in range(4):
                    out_v[qs * 16 + sub * 4 + i, :] = accs[i]
            return 0

        lax.fori_loop(0, mq // 16, qloop, 0)
        pltpu.sync_copy(out_v, out_hbm.at[pl.ds(base, mq), :])

    return sck


@functools.partial(jax.jit, static_argnames=("tm", "pc", "nsc"))
def _nn(x, pcd_t, tm=1024, pc=512, nsc=NSC):
    m, _ = x.shape
    _, n = pcd_t.shape
    ntc = n - nsc
    pall_sc, pall8 = pl.pallas_call(
        functools.partial(_pack_kernel, nsc=nsc),
        in_specs=[pl.BlockSpec((3, n), lambda: (0, 0))],
        out_specs=[pl.BlockSpec((4, nsc), lambda: (0, 0)),
                   pl.BlockSpec((4, 8, ntc), lambda: (0, 0, 0))],
        out_shape=[jax.ShapeDtypeStruct((4, nsc), jnp.float32),
                   jax.ShapeDtypeStruct((4, 8, ntc), jnp.float32)],
    )(pcd_t)

    xT4 = jnp.pad(x.T, ((0, 1), (0, 0)))
    usc = _make_sc_kernel(m, nsc)(xT4, pall_sc)

    utc = pl.pallas_call(
        functools.partial(_nn_tc_kernel, pc=pc, n=ntc, tm=tm),
        grid=(m // tm,),
        in_specs=[
            pl.BlockSpec((tm, 3), lambda i: (i, 0)),
            pl.BlockSpec((4, 8, ntc), lambda i: (0, 0, 0)),
        ],
        out_specs=pl.BlockSpec((tm, 1), lambda i: (i, 0)),
        out_shape=jax.ShapeDtypeStruct((m, 1), jnp.float32),
        scratch_shapes=[pltpu.VMEM((tm, 128), jnp.float32),
                        pltpu.VMEM((tm, 128), jnp.float32),
                        pltpu.VMEM((tm, 128), jnp.float32),
                        pltpu.VMEM((tm, 128), jnp.float32)],
    )(x, pall8)

    return pl.pallas_call(
        _combine_kernel,
        grid=(m // tm,),
        in_specs=[
            pl.BlockSpec((tm, 3), lambda i: (i, 0)),
            pl.BlockSpec((tm, 1), lambda i: (i, 0)),
            pl.BlockSpec((tm, 16), lambda i: (i, 0)),
        ],
        out_specs=pl.BlockSpec((tm, 1), lambda i: (i, 0)),
        out_shape=jax.ShapeDtypeStruct((m, 1), jnp.float32),
    )(x, utc, usc)


def kernel(x, pcd):
    return _nn(x, pcd.T)


# speedup vs baseline: 1.3012x; 1.3012x over previous
import functools

import jax
import jax.numpy as jnp
from jax import lax
from jax.experimental import pallas as pl
from jax.experimental.pallas import tpu as pltpu
from jax.experimental.pallas import tpu_sc as plsc

RADIUS = 0.05
NSC = 16384


def _pack_kernel(p_ref, osc_ref, otc_ref, *, nsc):
    pt = p_ref[...]
    pnn = -0.5 * (pt[0:1, :] * pt[0:1, :] + pt[1:2, :] * pt[1:2, :]
                  + pt[2:3, :] * pt[2:3, :])
    pk = jnp.concatenate([pt, pnn], axis=0)
    osc_ref[...] = pk[:, :nsc]
    otc_ref[...] = jnp.broadcast_to(pk[:, None, nsc:], otc_ref.shape)


def _nn_tc_kernel(x_ref, pall_ref, o_ref, xb0_ref, xb1_ref, xb2_ref,
                  acc_ref, *, pc, n, tm):
    xt = x_ref[...]
    xb0_ref[...] = jnp.broadcast_to(xt[:, 0:1], (tm, 128))
    xb1_ref[...] = jnp.broadcast_to(xt[:, 1:2], (tm, 128))
    xb2_ref[...] = jnp.broadcast_to(xt[:, 2:3], (tm, 128))
    acc_ref[...] = jnp.full((tm, 128), -jnp.inf, jnp.float32)

    def ploop(k, _):
        base = k * pc
        p0 = pall_ref[0, :, pl.ds(base, pc)]
        p1 = pall_ref[1, :, pl.ds(base, pc)]
        p2 = pall_ref[2, :, pl.ds(base, pc)]
        pnn = pall_ref[3, :, pl.ds(base, pc)]

        def qloop(qb, _):
            r = qb * 8
            x0 = xb0_ref[pl.ds(r, 8), :]
            x1 = xb1_ref[pl.ds(r, 8), :]
            x2 = xb2_ref[pl.ds(r, 8), :]
            m = None
            for j in range(pc // 128):
                s = slice(j * 128, (j + 1) * 128)
                u = x0 * p0[:, s] + (x1 * p1[:, s]
                                     + (x2 * p2[:, s] + pnn[:, s]))
                m = u if m is None else jnp.maximum(m, u)
            acc_ref[pl.ds(r, 8), :] = jnp.maximum(acc_ref[pl.ds(r, 8), :], m)
            return 0

        lax.fori_loop(0, tm // 8, qloop, 0, unroll=8)
        return 0

    lax.fori_loop(0, n // pc, ploop, 0)
    o_ref[:, 0] = jnp.max(acc_ref[...], axis=1)


def _combine_kernel(x_ref, utc_ref, usc_ref, o_ref):
    xt = x_ref[...]
    xn = (xt[:, 0] * xt[:, 0] + xt[:, 1] * xt[:, 1] + xt[:, 2] * xt[:, 2])
    u = jnp.maximum(utc_ref[:, 0], jnp.max(usc_ref[...], axis=1))
    d2 = jnp.maximum(xn - 2.0 * u, 0.0)
    o_ref[:, 0] = jnp.sqrt(d2) - RADIUS


def _make_sc_kernel(m, nsc):
    mq = m // 32
    mesh = plsc.VectorSubcoreMesh(core_axis_name="c", subcore_axis_name="s")

    @functools.partial(
        pl.kernel, mesh=mesh,
        out_type=jax.ShapeDtypeStruct((m, 16), jnp.float32),
        scratch_types=[
            pltpu.VMEM((4, mq), jnp.float32),
            pltpu.VMEM((4, nsc), jnp.float32),
            pltpu.VMEM((mq, 16), jnp.float32),
        ],
    )
    def sck(x_hbm, p_hbm, out_hbm, x_v, p_v, out_v):
        wid = lax.axis_index("s") * 2 + lax.axis_index("c")
        base = wid * mq
        pltpu.sync_copy(x_hbm.at[:, pl.ds(base, mq)], x_v)
        pltpu.sync_copy(p_hbm, p_v)

        def qloop(qs, _):
            q = qs * 16
            x0v = x_v[0, pl.ds(q, 16)]
            x1v = x_v[1, pl.ds(q, 16)]
            x2v = x_v[2, pl.ds(q, 16)]
            for sub in range(4):
                xs = []
                for i in range(4):
                    e = sub * 4 + i
                    xs.append((jnp.broadcast_to(x0v[e], (16,)),
                               jnp.broadcast_to(x1v[e], (16,)),
                               jnp.broadcast_to(x2v[e], (16,))))

                def jloop(j, accs):
                    c = j * 16
                    p0 = p_v[0, pl.ds(c, 16)]
                    p1 = p_v[1, pl.ds(c, 16)]
                    p2 = p_v[2, pl.ds(c, 16)]
                    pnn = p_v[3, pl.ds(c, 16)]
                    return tuple(
                        jnp.maximum(a, b0 * p0 + (b1 * p1 + (b2 * p2 + pnn)))
                        for a, (b0, b1, b2) in zip(accs, xs))

                neg = jnp.full((16,), -jnp.inf, jnp.float32)
                accs = lax.fori_loop(0, nsc // 16, jloop,
                                     (neg, neg, neg, neg))
                for i in range(4):
                    out_v[qs * 16 + sub * 4 + i, :] = accs[i]
            return 0

        lax.fori_loop(0, mq // 16, qloop, 0)
        pltpu.sync_copy(out_v, out_hbm.at[pl.ds(base, mq), :])

    return sck


@functools.partial(jax.jit, static_argnames=("tm", "pc", "nsc"))
def _nn(x, pcd_t, tm=1024, pc=1536, nsc=NSC):
    m, _ = x.shape
    _, n = pcd_t.shape
    ntc = n - nsc
    pall_sc, pall8 = pl.pallas_call(
        functools.partial(_pack_kernel, nsc=nsc),
        in_specs=[pl.BlockSpec((3, n), lambda: (0, 0))],
        out_specs=[pl.BlockSpec((4, nsc), lambda: (0, 0)),
                   pl.BlockSpec((4, 8, ntc), lambda: (0, 0, 0))],
        out_shape=[jax.ShapeDtypeStruct((4, nsc), jnp.float32),
                   jax.ShapeDtypeStruct((4, 8, ntc), jnp.float32)],
    )(pcd_t)

    xT4 = jnp.pad(x.T, ((0, 1), (0, 0)))
    usc = _make_sc_kernel(m, nsc)(xT4, pall_sc)

    utc = pl.pallas_call(
        functools.partial(_nn_tc_kernel, pc=pc, n=ntc, tm=tm),
        grid=(m // tm,),
        in_specs=[
            pl.BlockSpec((tm, 3), lambda i: (i, 0)),
            pl.BlockSpec((4, 8, ntc), lambda i: (0, 0, 0)),
        ],
        out_specs=pl.BlockSpec((tm, 1), lambda i: (i, 0)),
        out_shape=jax.ShapeDtypeStruct((m, 1), jnp.float32),
        scratch_shapes=[pltpu.VMEM((tm, 128), jnp.float32),
                        pltpu.VMEM((tm, 128), jnp.float32),
                        pltpu.VMEM((tm, 128), jnp.float32),
                        pltpu.VMEM((tm, 128), jnp.float32)],
    )(x, pall8)

    return pl.pallas_call(
        _combine_kernel,
        grid=(m // tm,),
        in_specs=[
            pl.BlockSpec((tm, 3), lambda i: (i, 0)),
            pl.BlockSpec((tm, 1), lambda i: (i, 0)),
            pl.BlockSpec((tm, 16), lambda i: (i, 0)),
        ],
        out_specs=pl.BlockSpec((tm, 1), lambda i: (i, 0)),
        out_shape=jax.ShapeDtypeStruct((m, 1), jnp.float32),
    )(x, utc, usc)


def kernel(x, pcd):
    return _nn(x, pcd.T)


# speedup vs baseline: 1.3111x; 1.0076x over previous
import functools

import jax
import jax.numpy as jnp
from jax import lax
from jax.experimental import pallas as pl
from jax.experimental.pallas import tpu as pltpu
from jax.experimental.pallas import tpu_sc as plsc

RADIUS = 0.05
NSC = 16384


def _pack_kernel(p_ref, osc_ref, otc_ref, *, nsc):
    pt = p_ref[...]
    pnn = -0.5 * (pt[0:1, :] * pt[0:1, :] + pt[1:2, :] * pt[1:2, :]
                  + pt[2:3, :] * pt[2:3, :])
    pk = jnp.concatenate([pt, pnn], axis=0)
    osc_ref[...] = pk[:, :nsc]
    otc_ref[...] = jnp.broadcast_to(pk[:, None, nsc:], otc_ref.shape)


def _nn_tc_kernel(x_ref, pall_ref, o_ref, xb0_ref, xb1_ref, xb2_ref,
                  acc_ref, *, pc, n, tm):
    xt = x_ref[...]
    xb0_ref[...] = jnp.broadcast_to(xt[:, 0:1], (tm, 128))
    xb1_ref[...] = jnp.broadcast_to(xt[:, 1:2], (tm, 128))
    xb2_ref[...] = jnp.broadcast_to(xt[:, 2:3], (tm, 128))
    acc_ref[...] = jnp.full((tm, 128), -jnp.inf, jnp.float32)

    def ploop(k, _):
        base = k * pc
        p0 = pall_ref[0, :, pl.ds(base, pc)]
        p1 = pall_ref[1, :, pl.ds(base, pc)]
        p2 = pall_ref[2, :, pl.ds(base, pc)]
        pnn = pall_ref[3, :, pl.ds(base, pc)]

        def qloop(qb, _):
            r = qb * 8
            x0 = xb0_ref[pl.ds(r, 8), :]
            x1 = xb1_ref[pl.ds(r, 8), :]
            x2 = xb2_ref[pl.ds(r, 8), :]
            m = None
            for j in range(pc // 128):
                s = slice(j * 128, (j + 1) * 128)
                u = x0 * p0[:, s] + (x1 * p1[:, s]
                                     + (x2 * p2[:, s] + pnn[:, s]))
                m = u if m is None else jnp.maximum(m, u)
            acc_ref[pl.ds(r, 8), :] = jnp.maximum(acc_ref[pl.ds(r, 8), :], m)
            return 0

        lax.fori_loop(0, tm // 8, qloop, 0, unroll=8)
        return 0

    lax.fori_loop(0, n // pc, ploop, 0)
    o_ref[:, 0] = jnp.max(acc_ref[...], axis=1)


def _combine_kernel(x_ref, utc_ref, usc_ref, o_ref):
    xt = x_ref[...]
    xn = (xt[:, 0] * xt[:, 0] + xt[:, 1] * xt[:, 1] + xt[:, 2] * xt[:, 2])
    u = jnp.maximum(utc_ref[:, 0], jnp.max(usc_ref[...], axis=1))
    d2 = jnp.maximum(xn - 2.0 * u, 0.0)
    o_ref[:, 0] = jnp.sqrt(d2) - RADIUS


def _make_sc_kernel(m, nsc):
    mq = m // 32
    mesh = plsc.VectorSubcoreMesh(core_axis_name="c", subcore_axis_name="s")

    @functools.partial(
        pl.kernel, mesh=mesh,
        out_type=jax.ShapeDtypeStruct((m, 16), jnp.float32),
        scratch_types=[
            pltpu.VMEM((4, mq), jnp.float32),
            pltpu.VMEM((4, nsc), jnp.float32),
            pltpu.VMEM((mq, 16), jnp.float32),
        ],
    )
    def sck(x_hbm, p_hbm, out_hbm, x_v, p_v, out_v):
        wid = lax.axis_index("s") * 2 + lax.axis_index("c")
        base = wid * mq
        pltpu.sync_copy(x_hbm.at[:, pl.ds(base, mq)], x_v)
        pltpu.sync_copy(p_hbm, p_v)

        def qloop(qs, _):
            q = qs * 16
            x0v = x_v[0, pl.ds(q, 16)]
            x1v = x_v[1, pl.ds(q, 16)]
            x2v = x_v[2, pl.ds(q, 16)]
            for sub in range(4):
                xs = []
                for i in range(4):
                    e = sub * 4 + i
                    xs.append((jnp.broadcast_to(x0v[e], (16,)),
                               jnp.broadcast_to(x1v[e], (16,)),
                               jnp.broadcast_to(x2v[e], (16,))))

                def jloop(j, accs):
                    c = j * 16
                    p0 = p_v[0, pl.ds(c, 16)]
                    p1 = p_v[1, pl.ds(c, 16)]
                    p2 = p_v[2, pl.ds(c, 16)]
                    pnn = p_v[3, pl.ds(c, 16)]
                    return tuple(
                        jnp.maximum(a, b0 * p0 + (b1 * p1 + (b2 * p2 + pnn)))
                        for a, (b0, b1, b2) in zip(accs, xs))

                neg = jnp.full((16,), -jnp.inf, jnp.float32)
                accs = lax.fori_loop(0, nsc // 16, jloop,
                                     (neg, neg, neg, neg))
                for i in range(4):
                    out_v[qs * 16 + sub * 4 + i, :] = accs[i]
            return 0

        lax.fori_loop(0, mq // 16, qloop, 0)
        pltpu.sync_copy(out_v, out_hbm.at[pl.ds(base, mq), :])

    return sck


@functools.partial(jax.jit, static_argnames=("tm", "pc", "nsc"))
def _nn(x, pcd_t, tm=2048, pc=1536, nsc=NSC):
    m, _ = x.shape
    _, n = pcd_t.shape
    ntc = n - nsc
    pall_sc, pall8 = pl.pallas_call(
        functools.partial(_pack_kernel, nsc=nsc),
        in_specs=[pl.BlockSpec((3, n), lambda: (0, 0))],
        out_specs=[pl.BlockSpec((4, nsc), lambda: (0, 0)),
                   pl.BlockSpec((4, 8, ntc), lambda: (0, 0, 0))],
        out_shape=[jax.ShapeDtypeStruct((4, nsc), jnp.float32),
                   jax.ShapeDtypeStruct((4, 8, ntc), jnp.float32)],
    )(pcd_t)

    xT4 = jnp.pad(x.T, ((0, 1), (0, 0)))
    usc = _make_sc_kernel(m, nsc)(xT4, pall_sc)

    utc = pl.pallas_call(
        functools.partial(_nn_tc_kernel, pc=pc, n=ntc, tm=tm),
        grid=(m // tm,),
        in_specs=[
            pl.BlockSpec((tm, 3), lambda i: (i, 0)),
            pl.BlockSpec((4, 8, ntc), lambda i: (0, 0, 0)),
        ],
        out_specs=pl.BlockSpec((tm, 1), lambda i: (i, 0)),
        out_shape=jax.ShapeDtypeStruct((m, 1), jnp.float32),
        scratch_shapes=[pltpu.VMEM((tm, 128), jnp.float32),
                        pltpu.VMEM((tm, 128), jnp.float32),
                        pltpu.VMEM((tm, 128), jnp.float32),
                        pltpu.VMEM((tm, 128), jnp.float32)],
    )(x, pall8)

    return pl.pallas_call(
        _combine_kernel,
        grid=(m // tm,),
        in_specs=[
            pl.BlockSpec((tm, 3), lambda i: (i, 0)),
            pl.BlockSpec((tm, 1), lambda i: (i, 0)),
            pl.BlockSpec((tm, 16), lambda i: (i, 0)),
        ],
        out_specs=pl.BlockSpec((tm, 1), lambda i: (i, 0)),
        out_shape=jax.ShapeDtypeStruct((m, 1), jnp.float32),
    )(x, utc, usc)


def kernel(x, pcd):
    return _nn(x, pcd.T)
